# baseline (speedup 1.0000x reference)
import functools

import jax
import jax.numpy as jnp
from jax import lax
from jax.experimental import pallas as pl
from jax.experimental.pallas import tpu as pltpu
from jax.experimental.pallas import tpu_sc as plsc

DIM = 32
HIST = 200
NUM_CORES = 2
NUM_SUBCORES = 16
NUM_WORKERS = NUM_CORES * NUM_SUBCORES
ROWS_PER_CHUNK = 8
SPLITS = ((0, 104), (104, 96))
NBUF = 2


@jax.jit
def _sc_embedding_gather(x, table):
    batch = x.shape[0]
    rows_per_worker = batch // NUM_WORKERS
    n_chunks = rows_per_worker // ROWS_PER_CHUNK
    assert n_chunks % NBUF == 0 and n_chunks // NBUF >= 2
    mesh = plsc.VectorSubcoreMesh(core_axis_name="c", subcore_axis_name="s")

    @functools.partial(
        pl.kernel,
        mesh=mesh,
        out_type=jax.ShapeDtypeStruct((batch, HIST, DIM), jnp.float32),
        scratch_types=[
            pltpu.VMEM((NBUF, ROWS_PER_CHUNK, HIST), jnp.int32),
            pltpu.VMEM((NBUF, ROWS_PER_CHUNK, HIST, DIM), jnp.float32),
            pltpu.SemaphoreType.DMA,
            pltpu.SemaphoreType.DMA,
            pltpu.SemaphoreType.DMA,
            pltpu.SemaphoreType.DMA,
            pltpu.SemaphoreType.DMA,
            pltpu.SemaphoreType.DMA,
        ],
        compiler_params=pltpu.CompilerParams(use_tc_tiling_on_sc=False),
    )
    def k(idx_hbm, table_hbm, out_hbm, idx_v, rows_v,
          gat0, gat1, out0, out1, lidx0, lidx1):
        gat = [gat0, gat1]
        osem = [out0, out1]
        isem = [lidx0, lidx1]
        wid = lax.axis_index("s") * NUM_CORES + lax.axis_index("c")
        base = wid * rows_per_worker

        def load_idx(i, b):
            pltpu.async_copy(
                idx_hbm.at[pl.ds(base + i * ROWS_PER_CHUNK, ROWS_PER_CHUNK)],
                idx_v.at[b], isem[b])

        def wait_idx(b):
            pltpu.make_async_copy(
                idx_hbm.at[pl.ds(base, ROWS_PER_CHUNK)], idx_v.at[b],
                isem[b]).wait()

        def fire_gathers(i, b):
            for r in range(ROWS_PER_CHUNK):
                for off, size in SPLITS:
                    sl = pl.ds(off, size)
                    pltpu.async_copy(
                        table_hbm.at[idx_v.at[b, r, sl]],
                        rows_v.at[b, r, sl], gat[b])

        def wait_gathers(b):
            for r in range(ROWS_PER_CHUNK):
                for off, size in SPLITS:
                    sl = pl.ds(off, size)
                    pltpu.make_async_copy(
                        table_hbm.at[idx_v.at[b, r, sl]],
                        rows_v.at[b, r, sl], gat[b]).wait()

        def store_out(i, b):
            pltpu.async_copy(
                rows_v.at[b],
                out_hbm.at[pl.ds(base + i * ROWS_PER_CHUNK, ROWS_PER_CHUNK)],
                osem[b])

        def wait_out(b):
            pltpu.make_async_copy(
                rows_v.at[b], out_hbm.at[pl.ds(base, ROWS_PER_CHUNK)],
                osem[b]).wait()

        for b in range(NBUF):
            load_idx(b, b)
        for b in range(NBUF):
            wait_idx(b)
            fire_gathers(b, b)
        for b in range(NBUF):
            wait_gathers(b)
            store_out(b, b)
            load_idx(b + NBUF, b)

        def body(g, carry):
            i0 = g * NBUF
            for b in range(NBUF):
                wait_idx(b)
                wait_out(b)
                fire_gathers(i0 + b, b)
            for b in range(NBUF):
                wait_gathers(b)
                store_out(i0 + b, b)
                load_idx(i0 + b + NBUF, b)
            return carry

        lax.fori_loop(1, n_chunks // NBUF - 1, body, 0)

        i0 = n_chunks - NBUF
        for b in range(NBUF):
            wait_idx(b)
            wait_out(b)
            fire_gathers(i0 + b, b)
        for b in range(NBUF):
            wait_gathers(b)
            store_out(i0 + b, b)
        for b in range(NBUF):
            wait_out(b)

    return k(x, table)


def kernel(x, table):
    return _sc_embedding_gather(x.astype(jnp.int32), table)


# speedup vs baseline: 1.6171x; 1.6171x over previous
import functools

import jax
import jax.numpy as jnp
from jax import lax
from jax.experimental import pallas as pl
from jax.experimental.pallas import tpu as pltpu
from jax.experimental.pallas import tpu_sc as plsc

DIM = 32
HIST = 200
BATCH = 16384
NUM_CORES = 2
NUM_SUBCORES = 16
NUM_WORKERS = NUM_CORES * NUM_SUBCORES
TT = HIST // 8
KT = BATCH // 128
K_PER_W = KT // NUM_WORKERS
N_UNITS = TT * K_PER_W
LPAD = 129


@jax.jit
def _sc_embedding_gather(x4, table):
    mesh = plsc.VectorSubcoreMesh(core_axis_name="c", subcore_axis_name="s")

    @functools.partial(
        pl.kernel,
        mesh=mesh,
        out_type=jax.ShapeDtypeStruct((HIST, DIM // 8, KT, 8, 128), jnp.float32),
        scratch_types=[
            pltpu.VMEM((2, 1024), jnp.int32),
            pltpu.VMEM((2, 4, 128, DIM), jnp.float32),
            pltpu.VMEM((2, 4, DIM // 8, 8, LPAD), jnp.float32),
            pltpu.SemaphoreType.DMA,
            pltpu.SemaphoreType.DMA,
            pltpu.SemaphoreType.DMA,
            pltpu.SemaphoreType.DMA,
            pltpu.SemaphoreType.DMA,
            pltpu.SemaphoreType.DMA,
        ],
        compiler_params=pltpu.CompilerParams(
            use_tc_tiling_on_sc=False, needs_layout_passes=False),
    )
    def k(x4_hbm, table_hbm, out_hbm, idx_v, rows_v, trans_v,
          isem0, isem1, gsem0, gsem1, ssem0, ssem1):
        isem = [isem0, isem1]
        gsem = [gsem0, gsem1]
        ssem = [ssem0, ssem1]
        wid = lax.axis_index("s") * NUM_CORES + lax.axis_index("c")
        iota = lax.iota(jnp.int32, 16)
        jv = [iota >> 3, 2 + (iota >> 3)]
        sv = iota & 7

        def unit_tt_k(u):
            return u // K_PER_W, wid * K_PER_W + (u % K_PER_W)

        def load_idx(u, bu):
            tt, kk = unit_tt_k(u)
            pltpu.async_copy(x4_hbm.at[tt, kk], idx_v.at[bu], isem[bu])

        def wait_idx(bu):
            pltpu.make_async_copy(x4_hbm.at[0, 0], idx_v.at[bu], isem[bu]).wait()

        def fire_gathers(bu, h):
            for ts in range(4):
                sl = pl.ds((4 * h + ts) * 128, 128)
                pltpu.async_copy(
                    table_hbm.at[idx_v.at[bu, sl]], rows_v.at[h, ts], gsem[h])

        def wait_gathers(bu, h):
            for ts in range(4):
                sl = pl.ds((4 * h + ts) * 128, 128)
                pltpu.make_async_copy(
                    table_hbm.at[idx_v.at[bu, sl]], rows_v.at[h, ts],
                    gsem[h]).wait()

        def transpose(h):
            def tbody(l, carry):
                lv = jnp.full((16,), l, jnp.int32)
                for ts in range(4):
                    for di, d0 in enumerate((0, 16)):
                        vals = rows_v[h, ts, l, pl.ds(d0, 16)]
                        plsc.store_scatter(
                            trans_v.at[h, ts], [jv[di], sv, lv], vals)
                return carry
            lax.fori_loop(0, 128, tbody, 0)

        def fire_stores(u, h):
            tt, kk = unit_tt_k(u)
            for ts in range(4):
                t = tt * 8 + 4 * h + ts
                pltpu.async_copy(
                    trans_v.at[h, ts, :, :, pl.ds(0, 128)],
                    out_hbm.at[t, :, kk], ssem[h])

        def wait_stores(h):
            for ts in range(4):
                pltpu.make_async_copy(
                    trans_v.at[h, ts, :, :, pl.ds(0, 128)],
                    out_hbm.at[0, :, 0], ssem[h]).wait()

        def unit_body(u, bu, first, last):
            wait_idx(bu)
            fire_gathers(bu, 0)
            fire_gathers(bu, 1)
            for h in range(2):
                wait_gathers(bu, h)
                if not first:
                    wait_stores(h)
                transpose(h)
                fire_stores(u, h)
            if not last:
                load_idx(u + 2, bu)

        load_idx(0, 0)
        load_idx(1, 1)
        unit_body(0, 0, True, False)
        unit_body(1, 1, False, False)

        def body(g, carry):
            unit_body(2 * g, 0, False, False)
            unit_body(2 * g + 1, 1, False, False)
            return carry

        lax.fori_loop(1, N_UNITS // 2 - 1, body, 0)

        unit_body(N_UNITS - 2, 0, False, True)
        unit_body(N_UNITS - 1, 1, False, True)
        for h in range(2):
            wait_stores(h)

    return k(x4, table)


def kernel(x, table):
    x4 = (
        x.astype(jnp.int32).T
        .reshape(TT, 8, KT, 128)
        .transpose(0, 2, 1, 3)
        .reshape(TT, KT, 1024)
    )
    out5 = _sc_embedding_gather(x4, table)
    return out5.transpose(2, 4, 0, 1, 3).reshape(BATCH, HIST, DIM)
